# baseline (speedup 1.0000x reference)
import jax
import jax.numpy as jnp
from jax.experimental import pallas as pl

_H = 128
_NG = 128
_REL_LIST = [
    ("ei_access", "P", "F"), ("ei_rev_access", "F", "P"),
    ("ei_same_as", "P", "F"), ("ei_rev_same_as", "F", "P"),
    ("ei_bind", "P", "Po"), ("ei_rev_bind", "Po", "P"),
    ("ei_session", "Po", "Po"), ("ei_rev_session", "Po", "Po"),
    ("ei_create", "P", "P"), ("ei_rev_create", "P", "P"),
]


def _mm_kern(x_ref, w_ref, o_ref):
    o_ref[...] = jnp.dot(x_ref[...], w_ref[...],
                         preferred_element_type=jnp.float32)


def _matmul(x, w):
    n, k = x.shape[0], w.shape[1]
    tn = 1000
    grid = (n // tn,)
    return pl.pallas_call(
        _mm_kern,
        grid=grid,
        in_specs=[
            pl.BlockSpec((tn, _H), lambda i: (i, 0)),
            pl.BlockSpec((_H, k), lambda i: (0, 0)),
        ],
        out_specs=pl.BlockSpec((tn, k), lambda i: (i, 0)),
        out_shape=jax.ShapeDtypeStruct((n, k), jnp.float32),
    )(x, w)


def _leaky(x, s):
    return jnp.where(x >= 0, x, s * x)


def _gat_edges(hs, hd, als, ald, ei, b):
    nd = hd.shape[0]
    src, dst = ei[0], ei[1]
    e = _leaky(als[src] + ald[dst], 0.2)
    m = jax.ops.segment_max(e, dst, num_segments=nd)
    m = jnp.where(jnp.isfinite(m), m, 0.0)
    ex = jnp.exp(e - m[dst])
    den = jax.ops.segment_sum(ex, dst, num_segments=nd)
    alpha = ex / (den[dst] + 1e-16)
    out = jax.ops.segment_sum(hs[src] * alpha[:, None], dst,
                              num_segments=nd)
    return out + b


def kernel(x_process, x_file, x_port, ei_access, ei_rev_access, ei_same_as,
           ei_rev_same_as, ei_bind, ei_rev_bind, ei_session, ei_rev_session,
           ei_create, ei_rev_create, batch_process, batch_file, batch_port,
           params):
    eidict = {
        "ei_access": ei_access, "ei_rev_access": ei_rev_access,
        "ei_same_as": ei_same_as, "ei_rev_same_as": ei_rev_same_as,
        "ei_bind": ei_bind, "ei_rev_bind": ei_rev_bind,
        "ei_session": ei_session, "ei_rev_session": ei_rev_session,
        "ei_create": ei_create, "ei_rev_create": ei_rev_create,
    }
    x = {"P": x_process, "F": x_file, "Po": x_port}
    for l in range(3):
        W = params["W_%d" % l]
        a_s = params["asrc_%d" % l]
        a_d = params["adst_%d" % l]
        b = params["b_%d" % l]
        h = {k: _matmul(x[k], W) for k in x}
        al_s = {k: jnp.sum(h[k] * a_s, axis=-1) for k in x}
        al_d = {k: jnp.sum(h[k] * a_d, axis=-1) for k in x}
        acc = {"P": [], "F": [], "Po": []}
        for name, st, dt in _REL_LIST:
            acc[dt].append(_gat_edges(h[st], h[dt], al_s[st], al_d[dt],
                                      eidict[name], b))
        x = {k: _leaky(jnp.mean(jnp.stack(acc[k], 0), 0), 0.01) for k in x}
    xa = jnp.concatenate([x["P"], x["F"], x["Po"]], 0)
    ba = jnp.concatenate([batch_process, batch_file, batch_port], 0)
    pooled = jax.ops.segment_sum(xa, ba, num_segments=_NG)
    pooled_p = jnp.pad(pooled, ((0, 1000 - _NG), (0, 0)))
    r1 = _matmul(pooled_p, params["lin1_W"])[: _NG] + params["lin1_b"]
    r1p = jnp.pad(r1, ((0, 1000 - _NG), (0, 0)))
    w2 = jnp.pad(params["lin2_W"], ((0, 0), (0, 126)))
    r2 = _matmul(r1p, w2)[: _NG, :2] + params["lin2_b"]
    return r2
